# baseline (speedup 1.0000x reference)
import functools

import jax
import jax.numpy as jnp
from jax import lax
from jax.experimental import pallas as pl
from jax.experimental.pallas import tpu as pltpu
from jax.experimental.pallas import tpu_sc as plsc

_B = 16384
_D = 64
_NC = 2
_NS = 16
_NW = _NC * _NS
_BPW = _B // _NW
_CHUNK = 128
_NCHUNKS = _BPW // _CHUNK


def _bpr_gather(user, pos_item, neg_item, user_emb, item_emb):
    mesh = plsc.VectorSubcoreMesh(
        core_axis_name="c", subcore_axis_name="s",
        num_cores=_NC, num_subcores=_NS)
    row = jax.ShapeDtypeStruct((_B, _D), jnp.float32)

    @functools.partial(
        pl.kernel,
        mesh=mesh,
        out_type=(row, row, row),
        compiler_params=pltpu.CompilerParams(use_tc_tiling_on_sc=False),
        scratch_types=[
            pltpu.VMEM((_BPW,), jnp.int32),
            pltpu.VMEM((_BPW,), jnp.int32),
            pltpu.VMEM((_BPW,), jnp.int32),
            pltpu.VMEM((_BPW, _D), jnp.float32),
            pltpu.VMEM((_BPW, _D), jnp.float32),
            pltpu.VMEM((_BPW, _D), jnp.float32),
            pltpu.SemaphoreType.DMA,
            pltpu.SemaphoreType.DMA,
            pltpu.SemaphoreType.DMA,
            pltpu.SemaphoreType.DMA,
        ],
    )
    def body(user_h, pos_h, neg_h, uemb_h, iemb_h,
             u_out, i_out, j_out,
             uidx, pidx, nidx, urows, irows, jrows,
             usem, psem, nsem, wsem):
        wid = lax.axis_index("s") * _NC + lax.axis_index("c")
        base = wid * _BPW
        pltpu.sync_copy(user_h.at[pl.ds(base, _BPW)], uidx)
        pltpu.sync_copy(pos_h.at[pl.ds(base, _BPW)], pidx)
        pltpu.sync_copy(neg_h.at[pl.ds(base, _BPW)], nidx)
        gathers = ([], [], [])
        for c in range(_NCHUNKS):
            sl = pl.ds(c * _CHUNK, _CHUNK)
            gathers[0].append(
                pltpu.async_copy(uemb_h.at[uidx.at[sl]], urows.at[sl], usem))
            gathers[1].append(
                pltpu.async_copy(iemb_h.at[pidx.at[sl]], irows.at[sl], psem))
            gathers[2].append(
                pltpu.async_copy(iemb_h.at[nidx.at[sl]], jrows.at[sl], nsem))
        writes = []
        for g, rows, out in ((gathers[0], urows, u_out),
                             (gathers[1], irows, i_out),
                             (gathers[2], jrows, j_out)):
            for h in g:
                h.wait()
            writes.append(
                pltpu.async_copy(rows, out.at[pl.ds(base, _BPW)], wsem))
        for h in writes:
            h.wait()

    return body(user, pos_item, neg_item, user_emb, item_emb)


def kernel(user, pos_item, neg_item, user_emb, item_emb):
    return _bpr_gather(user, pos_item, neg_item, user_emb, item_emb)
